# baseline (speedup 1.0000x reference)
import functools

import jax
import jax.numpy as jnp
from jax import lax
from jax.experimental import pallas as pl
from jax.experimental.pallas import tpu as pltpu
from jax.experimental.pallas import tpu_sc as plsc

NN = 100000
EE = 3200000
BB = 1024

NW = 32
N_PAD = 100352
RPT = N_PAD // 16
EPW = EE // NW

_MESH = dict(core_axis_name="c", subcore_axis_name="s")
_SC_PARAMS = pltpu.CompilerParams(use_tc_tiling_on_sc=False)


def _make_edge_pass(F, K):
    n = EPW // K
    assert n * K == EPW and K % 8 == 0

    mesh = plsc.VectorSubcoreMesh(**_MESH)

    @functools.partial(
        pl.kernel, mesh=mesh, compiler_params=_SC_PARAMS,
        out_type=jax.ShapeDtypeStruct((2 * N_PAD, F), jnp.float32),
        scratch_types=[
            pltpu.VMEM((2, K), jnp.int32),
            pltpu.VMEM((2, K), jnp.int32),
            pltpu.VMEM((K, F), jnp.float32),
            pltpu.VMEM((K, F), jnp.float32),
            pltpu.VMEM_SHARED((N_PAD, F), jnp.float32),
            pltpu.SemaphoreType.DMA,
            pltpu.SemaphoreType.DMA,
            pltpu.SemaphoreType.DMA,
            pltpu.SemaphoreType.DMA,
        ],
    )
    def k(table, ei, zrows, out,
          P0, P1, R0, R1, acc, si0, si1, sg0, sg1):
        c = lax.axis_index("c")
        s = lax.axis_index("s")
        wid = s * 2 + c
        row0 = s * RPT
        pltpu.sync_copy(zrows.at[pl.ds(row0, RPT)], acc.at[pl.ds(row0, RPT)])
        plsc.subcore_barrier()

        eb = wid * EPW

        def load(i, P, si):
            pltpu.async_copy(ei.at[:, pl.ds(eb + i * K, K)], P, si)

        def wait_load(P, si):
            pltpu.make_async_copy(ei.at[:, pl.ds(0, K)], P, si).wait()

        def body2(j, carry):
            i0 = 2 * j
            @pl.when(i0 + 1 < n)
            def _():
                wait_load(P1, si1)
            pltpu.make_async_copy(table.at[P0.at[0]], R0, sg0).wait()

            @pl.when(i0 + 1 < n)
            def _():
                pltpu.async_copy(table.at[P1.at[0]], R1, sg1)
            pltpu.sync_copy(R0, acc.at[P0.at[1]], add=True)

            @pl.when(i0 + 2 < n)
            def _():
                load(i0 + 2, P0, si0)

            @pl.when(i0 + 1 < n)
            def _():
                @pl.when(i0 + 2 < n)
                def _():
                    wait_load(P0, si0)
                pltpu.make_async_copy(table.at[P1.at[0]], R1, sg1).wait()

                @pl.when(i0 + 2 < n)
                def _():
                    pltpu.async_copy(table.at[P0.at[0]], R0, sg0)
                pltpu.sync_copy(R1, acc.at[P1.at[1]], add=True)

                @pl.when(i0 + 3 < n)
                def _():
                    load(i0 + 3, P1, si1)
            return carry

        load(0, P0, si0)
        load(1, P1, si1)
        wait_load(P0, si0)
        pltpu.async_copy(table.at[P0.at[0]], R0, sg0)
        lax.fori_loop(0, (n + 1) // 2, body2, 0)

        plsc.subcore_barrier()
        pltpu.sync_copy(acc.at[pl.ds(row0, RPT)],
                        out.at[pl.ds(c * N_PAD + row0, RPT)])

    return k


_edge16 = _make_edge_pass(16, 800)
_edge8 = _make_edge_pass(8, 2000)


def _make_deg():
    K = 10000
    n = EPW // K
    mesh = plsc.VectorSubcoreMesh(**_MESH)

    @functools.partial(
        pl.kernel, mesh=mesh, compiler_params=_SC_PARAMS,
        out_type=jax.ShapeDtypeStruct((2 * N_PAD,), jnp.float32),
        scratch_types=[
            pltpu.VMEM((K,), jnp.int32),
            pltpu.VMEM((K,), jnp.int32),
            pltpu.VMEM((K,), jnp.float32),
            pltpu.VMEM_SHARED((N_PAD,), jnp.float32),
            pltpu.SemaphoreType.DMA,
            pltpu.SemaphoreType.DMA,
        ],
    )
    def k(dsts, ones_hbm, zrows, out, dst_v, dst_w, ones_v, acc, si0, si1):
        c = lax.axis_index("c")
        s = lax.axis_index("s")
        wid = s * 2 + c
        row0 = s * RPT
        pltpu.sync_copy(ones_hbm, ones_v)
        pltpu.sync_copy(zrows.at[pl.ds(row0, RPT)], acc.at[pl.ds(row0, RPT)])
        plsc.subcore_barrier()

        ebase = wid * EPW

        def body2(j, carry):
            i0 = 2 * j
            pltpu.make_async_copy(dsts.at[pl.ds(0, K)], dst_v, si0).wait()
            pltpu.sync_copy(ones_v, acc.at[dst_v], add=True)

            @pl.when(i0 + 2 < n)
            def _():
                pltpu.async_copy(dsts.at[pl.ds(ebase + (i0 + 2) * K, K)],
                                 dst_v, si0)

            pltpu.make_async_copy(dsts.at[pl.ds(0, K)], dst_w, si1).wait()
            pltpu.sync_copy(ones_v, acc.at[dst_w], add=True)

            @pl.when(i0 + 3 < n)
            def _():
                pltpu.async_copy(dsts.at[pl.ds(ebase + (i0 + 3) * K, K)],
                                 dst_w, si1)
            return carry

        pltpu.async_copy(dsts.at[pl.ds(ebase, K)], dst_v, si0)
        pltpu.async_copy(dsts.at[pl.ds(ebase + K, K)], dst_w, si1)
        lax.fori_loop(0, n // 2, body2, 0)
        plsc.subcore_barrier()
        pltpu.sync_copy(acc.at[pl.ds(row0, RPT)],
                        out.at[pl.ds(c * N_PAD + row0, RPT)])

    return k


_deg = _make_deg()


def _make_pool():
    KP = 800
    ROWS_PER_W = 3200
    mesh = plsc.VectorSubcoreMesh(**_MESH)

    @functools.partial(
        pl.kernel, mesh=mesh, compiler_params=_SC_PARAMS,
        out_type=jax.ShapeDtypeStruct((2 * BB, 8), jnp.float32),
        scratch_types=[
            pltpu.VMEM((KP,), jnp.int32),
            pltpu.VMEM((KP, 8), jnp.float32),
            pltpu.VMEM_SHARED((BB, 8), jnp.float32),
        ],
    )
    def k(h2, batch, zrows, out, idx_v, rows_v, acc):
        c = lax.axis_index("c")
        s = lax.axis_index("s")
        wid = s * 2 + c
        r0 = s * (BB // 16)
        pltpu.sync_copy(zrows.at[pl.ds(r0, BB // 16)], acc.at[pl.ds(r0, BB // 16)])
        plsc.subcore_barrier()

        for j in range(ROWS_PER_W // KP):
            base = wid * ROWS_PER_W + j * KP

            @pl.when(base + KP <= NN)
            def _():
                pltpu.sync_copy(batch.at[pl.ds(base, KP)], idx_v)
                pltpu.sync_copy(h2.at[pl.ds(base, KP)], rows_v)
                pltpu.sync_copy(rows_v, acc.at[idx_v], add=True)

        plsc.subcore_barrier()
        pltpu.sync_copy(acc.at[pl.ds(r0, BB // 16)],
                        out.at[pl.ds(c * BB + r0, BB // 16)])

    return k


_pool = _make_pool()


def _tc1_body(deg_ref, xt_ref, w1t_ref, t1t_ref, dinv_ref):
    deg = deg_ref[0] + deg_ref[1] + 1.0
    dinv = lax.rsqrt(deg)[None, :]
    h = jnp.dot(w1t_ref[...], xt_ref[...], preferred_element_type=jnp.float32)
    t1t_ref[...] = h * dinv
    dinv_ref[...] = dinv


def _tc2_body(acct_ref, t1t_ref, dinv_ref, b1_ref, w2t_ref, t2t_ref):
    dinv = dinv_ref[...]
    h1 = jnp.tanh(dinv * (acct_ref[0] + acct_ref[1] + t1t_ref[...])
                  + b1_ref[...])
    t2t_ref[...] = jnp.dot(w2t_ref[...], h1,
                           preferred_element_type=jnp.float32) * dinv


def _tc3_body(acct_ref, t2t_ref, dinv_ref, b2_ref, h2t_ref):
    dinv = dinv_ref[...]
    h2t_ref[...] = jnp.tanh(dinv * (acct_ref[0] + acct_ref[1] + t2t_ref[...])
                            + b2_ref[...])


def _tc4_body(pooled_ref, add_ref, w3_ref, b3_ref, w4a_ref, w4b_ref, b4_ref,
              out_ref):
    pooled = pooled_ref[0] + pooled_ref[1]
    add_x = jnp.tanh(jnp.dot(add_ref[...], w3_ref[...],
                             preferred_element_type=jnp.float32) + b3_ref[...])
    out_ref[...] = (jnp.dot(pooled, w4a_ref[...], preferred_element_type=jnp.float32)
                    + jnp.dot(add_x, w4b_ref[...], preferred_element_type=jnp.float32)
                    + b4_ref[...])


def kernel(x, edge_index, batch, y, p, c, apf, wiener,
           W1, b1, W2, b2, W3, b3, W4, b4):
    src = edge_index[0]
    dst = edge_index[1]
    xt = jnp.pad(x, ((0, N_PAD - NN), (0, 0))).T

    z16 = jnp.zeros((N_PAD, 16), jnp.float32)
    z8 = jnp.zeros((N_PAD, 8), jnp.float32)
    z1 = jnp.zeros((N_PAD,), jnp.float32)
    zb = jnp.zeros((BB, 8), jnp.float32)
    ones = jnp.ones((10000,), jnp.float32)

    deg2 = _deg(dst, ones, z1).reshape(2, N_PAD)

    t1t, dinv = pl.pallas_call(
        _tc1_body,
        out_shape=[jax.ShapeDtypeStruct((16, N_PAD), jnp.float32),
                   jax.ShapeDtypeStruct((1, N_PAD), jnp.float32)],
    )(deg2, xt, W1.T)

    acc1 = _edge16(t1t.T, edge_index, z16)
    acc1t = jnp.transpose(acc1.reshape(2, N_PAD, 16), (0, 2, 1))

    t2t = pl.pallas_call(
        _tc2_body,
        out_shape=jax.ShapeDtypeStruct((8, N_PAD), jnp.float32),
    )(acc1t, t1t, dinv, b1[:, None], W2.T)

    acc2 = _edge8(t2t.T, edge_index, z8)
    acc2t = jnp.transpose(acc2.reshape(2, N_PAD, 8), (0, 2, 1))

    h2t = pl.pallas_call(
        _tc3_body,
        out_shape=jax.ShapeDtypeStruct((8, N_PAD), jnp.float32),
    )(acc2t, t2t, dinv, b2[:, None])

    pooled2 = _pool(h2t.T, batch, zb).reshape(2, BB, 8)

    additional = jnp.concatenate(
        [p[:, None], c[:, None], apf[:, None], wiener[:, None]], axis=1)

    out = pl.pallas_call(
        _tc4_body,
        out_shape=jax.ShapeDtypeStruct((BB, 1), jnp.float32),
    )(pooled2, additional, W3, b3, W4[:8], W4[8:], b4)

    return out


# speedup vs baseline: 1.0032x; 1.0032x over previous
import functools

import jax
import jax.numpy as jnp
from jax import lax
from jax.experimental import pallas as pl
from jax.experimental.pallas import tpu as pltpu
from jax.experimental.pallas import tpu_sc as plsc

NN = 100000
EE = 3200000
BB = 1024

NW = 32
N_PAD = 100352
RPT = N_PAD // 16
EPW = EE // NW

_MESH = dict(core_axis_name="c", subcore_axis_name="s")
_SC_PARAMS = pltpu.CompilerParams(use_tc_tiling_on_sc=False)


def _make_edge_pass(F, K):
    n = EPW // K
    assert n * K == EPW and K % 8 == 0

    mesh = plsc.VectorSubcoreMesh(**_MESH)

    @functools.partial(
        pl.kernel, mesh=mesh, compiler_params=_SC_PARAMS,
        out_type=jax.ShapeDtypeStruct((2 * N_PAD, F), jnp.float32),
        scratch_types=[
            pltpu.VMEM((K,), jnp.int32),
            pltpu.VMEM((K,), jnp.int32),
            pltpu.VMEM((K,), jnp.int32),
            pltpu.VMEM((K,), jnp.int32),
            pltpu.VMEM((K, F), jnp.float32),
            pltpu.VMEM((K, F), jnp.float32),
            pltpu.VMEM_SHARED((N_PAD, F), jnp.float32),
            pltpu.SemaphoreType.DMA,
            pltpu.SemaphoreType.DMA,
            pltpu.SemaphoreType.DMA,
            pltpu.SemaphoreType.DMA,
        ],
    )
    def k(table, srcs, dsts, zrows, out,
          S0, D0, S1, D1, R0, R1, acc, si0, si1, sg0, sg1):
        c = lax.axis_index("c")
        s = lax.axis_index("s")
        wid = s * 2 + c
        row0 = s * RPT
        pltpu.sync_copy(zrows.at[pl.ds(row0, RPT)], acc.at[pl.ds(row0, RPT)])
        plsc.subcore_barrier()

        eb = wid * EPW

        def load(i, S, D, si):
            pltpu.async_copy(srcs.at[pl.ds(eb + i * K, K)], S, si)
            pltpu.async_copy(dsts.at[pl.ds(eb + i * K, K)], D, si)

        def wait_load(S, D, si):
            pltpu.make_async_copy(srcs.at[pl.ds(0, K)], S, si).wait()
            pltpu.make_async_copy(dsts.at[pl.ds(0, K)], D, si).wait()

        def body2(j, carry):
            i0 = 2 * j
            @pl.when(i0 + 1 < n)
            def _():
                wait_load(S1, D1, si1)
            pltpu.make_async_copy(table.at[S0], R0, sg0).wait()

            @pl.when(i0 + 1 < n)
            def _():
                pltpu.async_copy(table.at[S1], R1, sg1)
            pltpu.sync_copy(R0, acc.at[D0], add=True)

            @pl.when(i0 + 2 < n)
            def _():
                load(i0 + 2, S0, D0, si0)

            @pl.when(i0 + 1 < n)
            def _():
                @pl.when(i0 + 2 < n)
                def _():
                    wait_load(S0, D0, si0)
                pltpu.make_async_copy(table.at[S1], R1, sg1).wait()

                @pl.when(i0 + 2 < n)
                def _():
                    pltpu.async_copy(table.at[S0], R0, sg0)
                pltpu.sync_copy(R1, acc.at[D1], add=True)

                @pl.when(i0 + 3 < n)
                def _():
                    load(i0 + 3, S1, D1, si1)
            return carry

        load(0, S0, D0, si0)
        load(1, S1, D1, si1)
        wait_load(S0, D0, si0)
        pltpu.async_copy(table.at[S0], R0, sg0)
        lax.fori_loop(0, (n + 1) // 2, body2, 0)

        plsc.subcore_barrier()
        pltpu.sync_copy(acc.at[pl.ds(row0, RPT)],
                        out.at[pl.ds(c * N_PAD + row0, RPT)])

    return k


_edge16 = _make_edge_pass(16, 800)
_edge8 = _make_edge_pass(8, 2000)


def _make_deg():
    K = 10000
    n = EPW // K
    mesh = plsc.VectorSubcoreMesh(**_MESH)

    @functools.partial(
        pl.kernel, mesh=mesh, compiler_params=_SC_PARAMS,
        out_type=jax.ShapeDtypeStruct((2 * N_PAD,), jnp.float32),
        scratch_types=[
            pltpu.VMEM((K,), jnp.int32),
            pltpu.VMEM((K,), jnp.int32),
            pltpu.VMEM((K,), jnp.float32),
            pltpu.VMEM_SHARED((N_PAD,), jnp.float32),
            pltpu.SemaphoreType.DMA,
            pltpu.SemaphoreType.DMA,
        ],
    )
    def k(dsts, ones_hbm, zrows, out, dst_v, dst_w, ones_v, acc, si0, si1):
        c = lax.axis_index("c")
        s = lax.axis_index("s")
        wid = s * 2 + c
        row0 = s * RPT
        pltpu.sync_copy(ones_hbm, ones_v)
        pltpu.sync_copy(zrows.at[pl.ds(row0, RPT)], acc.at[pl.ds(row0, RPT)])
        plsc.subcore_barrier()

        ebase = wid * EPW

        def body2(j, carry):
            i0 = 2 * j
            pltpu.make_async_copy(dsts.at[pl.ds(0, K)], dst_v, si0).wait()
            pltpu.sync_copy(ones_v, acc.at[dst_v], add=True)

            @pl.when(i0 + 2 < n)
            def _():
                pltpu.async_copy(dsts.at[pl.ds(ebase + (i0 + 2) * K, K)],
                                 dst_v, si0)

            pltpu.make_async_copy(dsts.at[pl.ds(0, K)], dst_w, si1).wait()
            pltpu.sync_copy(ones_v, acc.at[dst_w], add=True)

            @pl.when(i0 + 3 < n)
            def _():
                pltpu.async_copy(dsts.at[pl.ds(ebase + (i0 + 3) * K, K)],
                                 dst_w, si1)
            return carry

        pltpu.async_copy(dsts.at[pl.ds(ebase, K)], dst_v, si0)
        pltpu.async_copy(dsts.at[pl.ds(ebase + K, K)], dst_w, si1)
        lax.fori_loop(0, n // 2, body2, 0)
        plsc.subcore_barrier()
        pltpu.sync_copy(acc.at[pl.ds(row0, RPT)],
                        out.at[pl.ds(c * N_PAD + row0, RPT)])

    return k


_deg = _make_deg()


def _make_pool():
    KP = 800
    ROWS_PER_W = 3200
    mesh = plsc.VectorSubcoreMesh(**_MESH)

    @functools.partial(
        pl.kernel, mesh=mesh, compiler_params=_SC_PARAMS,
        out_type=jax.ShapeDtypeStruct((2 * BB, 8), jnp.float32),
        scratch_types=[
            pltpu.VMEM((KP,), jnp.int32),
            pltpu.VMEM((KP, 8), jnp.float32),
            pltpu.VMEM_SHARED((BB, 8), jnp.float32),
        ],
    )
    def k(h2, batch, zrows, out, idx_v, rows_v, acc):
        c = lax.axis_index("c")
        s = lax.axis_index("s")
        wid = s * 2 + c
        r0 = s * (BB // 16)
        pltpu.sync_copy(zrows.at[pl.ds(r0, BB // 16)], acc.at[pl.ds(r0, BB // 16)])
        plsc.subcore_barrier()

        for j in range(ROWS_PER_W // KP):
            base = wid * ROWS_PER_W + j * KP

            @pl.when(base + KP <= NN)
            def _():
                pltpu.sync_copy(batch.at[pl.ds(base, KP)], idx_v)
                pltpu.sync_copy(h2.at[pl.ds(base, KP)], rows_v)
                pltpu.sync_copy(rows_v, acc.at[idx_v], add=True)

        plsc.subcore_barrier()
        pltpu.sync_copy(acc.at[pl.ds(r0, BB // 16)],
                        out.at[pl.ds(c * BB + r0, BB // 16)])

    return k


_pool = _make_pool()


def _tc1_body(deg_ref, xt_ref, w1t_ref, t1t_ref, dinv_ref):
    deg = deg_ref[0] + deg_ref[1] + 1.0
    dinv = lax.rsqrt(deg)[None, :]
    h = jnp.dot(w1t_ref[...], xt_ref[...], preferred_element_type=jnp.float32)
    t1t_ref[...] = h * dinv
    dinv_ref[...] = dinv


def _tc2_body(acct_ref, t1t_ref, dinv_ref, b1_ref, w2t_ref, t2t_ref):
    dinv = dinv_ref[...]
    h1 = jnp.tanh(dinv * (acct_ref[0] + acct_ref[1] + t1t_ref[...])
                  + b1_ref[...])
    t2t_ref[...] = jnp.dot(w2t_ref[...], h1,
                           preferred_element_type=jnp.float32) * dinv


def _tc3_body(acct_ref, t2t_ref, dinv_ref, b2_ref, h2t_ref):
    dinv = dinv_ref[...]
    h2t_ref[...] = jnp.tanh(dinv * (acct_ref[0] + acct_ref[1] + t2t_ref[...])
                            + b2_ref[...])


def _tc4_body(pooled_ref, add_ref, w3_ref, b3_ref, w4a_ref, w4b_ref, b4_ref,
              out_ref):
    pooled = pooled_ref[0] + pooled_ref[1]
    add_x = jnp.tanh(jnp.dot(add_ref[...], w3_ref[...],
                             preferred_element_type=jnp.float32) + b3_ref[...])
    out_ref[...] = (jnp.dot(pooled, w4a_ref[...], preferred_element_type=jnp.float32)
                    + jnp.dot(add_x, w4b_ref[...], preferred_element_type=jnp.float32)
                    + b4_ref[...])


def kernel(x, edge_index, batch, y, p, c, apf, wiener,
           W1, b1, W2, b2, W3, b3, W4, b4):
    src = edge_index[0]
    dst = edge_index[1]
    xt = jnp.pad(x, ((0, N_PAD - NN), (0, 0))).T

    z16 = jnp.zeros((N_PAD, 16), jnp.float32)
    z8 = jnp.zeros((N_PAD, 8), jnp.float32)
    z1 = jnp.zeros((N_PAD,), jnp.float32)
    zb = jnp.zeros((BB, 8), jnp.float32)
    ones = jnp.ones((10000,), jnp.float32)

    deg2 = _deg(dst, ones, z1).reshape(2, N_PAD)

    t1t, dinv = pl.pallas_call(
        _tc1_body,
        out_shape=[jax.ShapeDtypeStruct((16, N_PAD), jnp.float32),
                   jax.ShapeDtypeStruct((1, N_PAD), jnp.float32)],
    )(deg2, xt, W1.T)

    acc1 = _edge16(t1t.T, src, dst, z16)
    acc1t = jnp.transpose(acc1.reshape(2, N_PAD, 16), (0, 2, 1))

    t2t = pl.pallas_call(
        _tc2_body,
        out_shape=jax.ShapeDtypeStruct((8, N_PAD), jnp.float32),
    )(acc1t, t1t, dinv, b1[:, None], W2.T)

    acc2 = _edge8(t2t.T, src, dst, z8)
    acc2t = jnp.transpose(acc2.reshape(2, N_PAD, 8), (0, 2, 1))

    h2t = pl.pallas_call(
        _tc3_body,
        out_shape=jax.ShapeDtypeStruct((8, N_PAD), jnp.float32),
    )(acc2t, t2t, dinv, b2[:, None])

    pooled2 = _pool(h2t.T, batch, zb).reshape(2, BB, 8)

    additional = jnp.concatenate(
        [p[:, None], c[:, None], apf[:, None], wiener[:, None]], axis=1)

    out = pl.pallas_call(
        _tc4_body,
        out_shape=jax.ShapeDtypeStruct((BB, 1), jnp.float32),
    )(pooled2, additional, W3, b3, W4[:8], W4[8:], b4)

    return out
